# baseline (speedup 1.0000x reference)
import functools

import jax
import jax.numpy as jnp
from jax import lax
from jax.experimental import pallas as pl
from jax.experimental.pallas import tpu as pltpu
from jax.experimental.pallas import tpu_sc as plsc

_PMIN = 0.1

_NC = 2
_NS = 16
_NW = _NC * _NS
_LANES = 16
_CHUNK = 16384
_UNROLL = 4


def _sc_mask_scale(total):
    per_worker = total // _NW
    n_chunks = per_worker // _CHUNK
    mesh = plsc.VectorSubcoreMesh(
        core_axis_name="c", subcore_axis_name="s",
        num_cores=_NC, num_subcores=_NS,
    )

    @functools.partial(
        pl.kernel,
        mesh=mesh,
        out_type=jax.ShapeDtypeStruct((total,), jnp.float32),
        scratch_types=[
            pltpu.VMEM((_CHUNK,), jnp.float32),
            pltpu.VMEM((_CHUNK,), jnp.float32),
        ],
    )
    def body(x_hbm, noise_hbm, out_hbm, xv, nv):
        p = jnp.float32(_PMIN)
        inv = jnp.float32(1.0) / (jnp.float32(1.0) - p)
        zero = jnp.zeros((_LANES,), jnp.float32)
        wid = lax.axis_index("s") * _NC + lax.axis_index("c")
        base = wid * per_worker

        def chunk_body(ci, carry):
            off = base + ci * _CHUNK
            pltpu.sync_copy(x_hbm.at[pl.ds(off, _CHUNK)], xv)
            pltpu.sync_copy(noise_hbm.at[pl.ds(off, _CHUNK)], nv)

            def vec_body(i, c):
                for u in range(_UNROLL):
                    s = pl.ds((i * _UNROLL + u) * _LANES, _LANES)
                    xi = xv[s]
                    ni = nv[s]
                    xv[s] = jnp.where(ni > p, xi * inv, zero)
                return c

            lax.fori_loop(0, _CHUNK // (_LANES * _UNROLL), vec_body, 0,
                          unroll=False)
            pltpu.sync_copy(xv, out_hbm.at[pl.ds(off, _CHUNK)])
            return carry

        lax.fori_loop(0, n_chunks, chunk_body, 0, unroll=False)

    return body


def kernel(x, noise):
    m, n = x.shape
    total = m * n
    out = _sc_mask_scale(total)(x.reshape(total), noise.reshape(total))
    return out.reshape(m, n)


# speedup vs baseline: 1.1099x; 1.1099x over previous
import functools

import jax
import jax.numpy as jnp
from jax import lax
from jax.experimental import pallas as pl
from jax.experimental.pallas import tpu as pltpu
from jax.experimental.pallas import tpu_sc as plsc

_PMIN = 0.1

_NC = 2
_NS = 16
_NW = _NC * _NS
_LANES = 16
_CHUNK = 16384
_UNROLL = 8


def _sc_mask_scale(total):
    per_worker = total // _NW
    n_chunks = per_worker // _CHUNK
    mesh = plsc.VectorSubcoreMesh(
        core_axis_name="c", subcore_axis_name="s",
        num_cores=_NC, num_subcores=_NS,
    )

    @functools.partial(
        pl.kernel,
        mesh=mesh,
        out_type=jax.ShapeDtypeStruct((total,), jnp.float32),
        scratch_types=[
            pltpu.VMEM((2, _CHUNK), jnp.float32),
            pltpu.VMEM((2, _CHUNK), jnp.float32),
            pltpu.VMEM((2, _CHUNK), jnp.float32),
            pltpu.SemaphoreType.DMA,
            pltpu.SemaphoreType.DMA,
            pltpu.SemaphoreType.DMA,
            pltpu.SemaphoreType.DMA,
        ],
    )
    def body(x_hbm, noise_hbm, out_hbm, xv, nv, ov,
             sem_in0, sem_in1, sem_out0, sem_out1):
        p = jnp.float32(_PMIN)
        inv = jnp.float32(1.0) / (jnp.float32(1.0) - p)
        zero = jnp.zeros((_LANES,), jnp.float32)
        wid = lax.axis_index("s") * _NC + lax.axis_index("c")
        base = wid * per_worker
        in_sems = (sem_in0, sem_in1)
        out_sems = (sem_out0, sem_out1)

        def start_in(ci):
            b = ci % 2
            off = base + ci * _CHUNK
            cx = pltpu.async_copy(x_hbm.at[pl.ds(off, _CHUNK)], xv.at[b],
                                  in_sems[b])
            cn = pltpu.async_copy(noise_hbm.at[pl.ds(off, _CHUNK)], nv.at[b],
                                  in_sems[b])
            return cx, cn

        def compute(b):
            def vec_body(i, c):
                for u in range(_UNROLL):
                    s = pl.ds((i * _UNROLL + u) * _LANES, _LANES)
                    xi = xv[b, s]
                    ni = nv[b, s]
                    ov[b, s] = jnp.where(ni > p, xi * inv, zero)
                return c
            lax.fori_loop(0, _CHUNK // (_LANES * _UNROLL), vec_body, 0,
                          unroll=False)

        pending_in = start_in(0)
        pending_out = [None, None]
        for ci in range(n_chunks):
            b = ci % 2
            nxt = None
            if ci + 1 < n_chunks:
                nxt = start_in(ci + 1)
            cx, cn = pending_in
            cx.wait()
            cn.wait()
            pending_in = nxt
            if pending_out[b] is not None:
                pending_out[b].wait()
            compute(b)
            off = base + ci * _CHUNK
            pending_out[b] = pltpu.async_copy(
                ov.at[b], out_hbm.at[pl.ds(off, _CHUNK)], out_sems[b])
        for po in pending_out:
            if po is not None:
                po.wait()

    return body


def kernel(x, noise):
    m, n = x.shape
    total = m * n
    out = _sc_mask_scale(total)(x.reshape(total), noise.reshape(total))
    return out.reshape(m, n)


# speedup vs baseline: 1.1162x; 1.0056x over previous
import functools

import jax
import jax.numpy as jnp
from jax import lax
from jax.experimental import pallas as pl
from jax.experimental.pallas import tpu as pltpu
from jax.experimental.pallas import tpu_sc as plsc

_PMIN = 0.1

_NC = 2
_NS = 16
_NW = _NC * _NS
_LANES = 16
_CHUNK = 16384
_UNROLL = 8


def _sc_mask_scale(total):
    per_worker = total // _NW
    n_chunks = per_worker // _CHUNK
    mesh = plsc.VectorSubcoreMesh(
        core_axis_name="c", subcore_axis_name="s",
        num_cores=_NC, num_subcores=_NS,
    )

    @functools.partial(
        pl.kernel,
        mesh=mesh,
        out_type=jax.ShapeDtypeStruct((total,), jnp.float32),
        scratch_types=[
            pltpu.VMEM((2, _CHUNK), jnp.float32),
            pltpu.VMEM((2, _CHUNK), jnp.float32),
            pltpu.VMEM((2, _CHUNK), jnp.float32),
            pltpu.SemaphoreType.DMA,
            pltpu.SemaphoreType.DMA,
            pltpu.SemaphoreType.DMA,
            pltpu.SemaphoreType.DMA,
        ],
    )
    def body(x_hbm, noise_hbm, out_hbm, xv, nv, ov,
             sem_in0, sem_in1, sem_out0, sem_out1):
        p = jnp.float32(_PMIN)
        inv = jnp.float32(1.0) / (jnp.float32(1.0) - p)
        zero = jnp.zeros((_LANES,), jnp.float32)
        wid = lax.axis_index("s") * _NC + lax.axis_index("c")
        base = wid * per_worker
        in_sems = (sem_in0, sem_in1)
        out_sems = (sem_out0, sem_out1)

        def start_in(ci):
            b = ci % 2
            off = base + ci * _CHUNK
            cx = pltpu.async_copy(x_hbm.at[pl.ds(off, _CHUNK)], xv.at[b],
                                  in_sems[b])
            cn = pltpu.async_copy(noise_hbm.at[pl.ds(off, _CHUNK)], nv.at[b],
                                  in_sems[b])
            return cx, cn

        def compute(b):
            @plsc.parallel_loop(0, _CHUNK, step=_LANES, unroll=_UNROLL)
            def _(i):
                s = pl.ds(i, _LANES)
                xi = xv[b, s]
                ni = nv[b, s]
                ov[b, s] = jnp.where(ni > p, xi * inv, zero)

        pending_in = start_in(0)
        pending_out = [None, None]
        for ci in range(n_chunks):
            b = ci % 2
            nxt = None
            if ci + 1 < n_chunks:
                nxt = start_in(ci + 1)
            cx, cn = pending_in
            cx.wait()
            cn.wait()
            pending_in = nxt
            if pending_out[b] is not None:
                pending_out[b].wait()
            compute(b)
            off = base + ci * _CHUNK
            pending_out[b] = pltpu.async_copy(
                ov.at[b], out_hbm.at[pl.ds(off, _CHUNK)], out_sems[b])
        for po in pending_out:
            if po is not None:
                po.wait()

    return body


def kernel(x, noise):
    m, n = x.shape
    total = m * n
    out = _sc_mask_scale(total)(x.reshape(total), noise.reshape(total))
    return out.reshape(m, n)


# speedup vs baseline: 2.4094x; 2.1585x over previous
import functools

import jax
import jax.numpy as jnp
from jax import lax
from jax.experimental import pallas as pl
from jax.experimental.pallas import tpu as pltpu
from jax.experimental.pallas import tpu_sc as plsc

_PMIN = 0.1

_NC = 2
_NS = 16
_NW = _NC * _NS
_LANES = 16
_ROWS = 8
_CCOLS = 2048
_UNROLL = 8


def _sc_mask_scale(m, n):
    cols_per_worker = n // 2
    n_chunks = cols_per_worker // _CCOLS
    mesh = plsc.VectorSubcoreMesh(
        core_axis_name="c", subcore_axis_name="s",
        num_cores=_NC, num_subcores=_NS,
    )

    @functools.partial(
        pl.kernel,
        mesh=mesh,
        out_type=jax.ShapeDtypeStruct((m, n), jnp.float32),
        scratch_types=[
            pltpu.VMEM((2, _ROWS, _CCOLS), jnp.float32),
            pltpu.VMEM((2, _ROWS, _CCOLS), jnp.float32),
            pltpu.VMEM((2, _ROWS, _CCOLS), jnp.float32),
            pltpu.SemaphoreType.DMA,
            pltpu.SemaphoreType.DMA,
            pltpu.SemaphoreType.DMA,
            pltpu.SemaphoreType.DMA,
        ],
        compiler_params=pltpu.CompilerParams(use_tc_tiling_on_sc=True),
    )
    def body(x_hbm, noise_hbm, out_hbm, xv, nv, ov,
             sem_in0, sem_in1, sem_out0, sem_out1):
        p = jnp.float32(_PMIN)
        inv = jnp.float32(1.0) / (jnp.float32(1.0) - p)
        zero = jnp.zeros((_LANES,), jnp.float32)
        wid = lax.axis_index("s") * _NC + lax.axis_index("c")
        row0 = (wid // 2) * _ROWS
        col0 = (wid % 2) * cols_per_worker
        in_sems = (sem_in0, sem_in1)
        out_sems = (sem_out0, sem_out1)

        def slab(ref, ci):
            c = col0 + ci * _CCOLS
            return ref.at[pl.ds(row0, _ROWS), pl.ds(c, _CCOLS)]

        def start_in(ci):
            b = ci % 2
            cx = pltpu.async_copy(slab(x_hbm, ci), xv.at[b], in_sems[b])
            cn = pltpu.async_copy(slab(noise_hbm, ci), nv.at[b], in_sems[b])
            return cx, cn

        def compute(b):
            for r in range(_ROWS):
                @plsc.parallel_loop(0, _CCOLS, step=_LANES, unroll=_UNROLL)
                def _(i):
                    s = pl.ds(i, _LANES)
                    ov[b, r, s] = jnp.where(nv[b, r, s] > p,
                                            xv[b, r, s] * inv, zero)

        pending_in = start_in(0)
        pending_out = [None, None]
        for ci in range(n_chunks):
            b = ci % 2
            nxt = None
            if ci + 1 < n_chunks:
                nxt = start_in(ci + 1)
            cx, cn = pending_in
            cx.wait()
            cn.wait()
            pending_in = nxt
            if pending_out[b] is not None:
                pending_out[b].wait()
            compute(b)
            pending_out[b] = pltpu.async_copy(ov.at[b], slab(out_hbm, ci),
                                              out_sems[b])
        for po in pending_out:
            if po is not None:
                po.wait()

    return body


def kernel(x, noise):
    m, n = x.shape
    return _sc_mask_scale(m, n)(x, noise)


# speedup vs baseline: 6.3385x; 2.6308x over previous
import jax
import jax.numpy as jnp
from jax.experimental import pallas as pl

_PMIN = 0.1
_ROWS_PER_BLOCK = 32


def _mask_scale_kernel(x_ref, noise_ref, out_ref):
    p = jnp.float32(_PMIN)
    inv = jnp.float32(1.0) / (jnp.float32(1.0) - p)
    x = x_ref[...]
    noise = noise_ref[...]
    out_ref[...] = jnp.where(noise > p, x * inv, jnp.float32(0.0))


def kernel(x, noise):
    m, n = x.shape
    grid = (m // _ROWS_PER_BLOCK,)
    spec = pl.BlockSpec((_ROWS_PER_BLOCK, n), lambda i: (i, 0))
    return pl.pallas_call(
        _mask_scale_kernel,
        grid=grid,
        in_specs=[spec, spec],
        out_specs=spec,
        out_shape=jax.ShapeDtypeStruct((m, n), jnp.float32),
    )(x, noise)
